# baseline (speedup 1.0000x reference)
import jax
import jax.numpy as jnp
from jax import lax
from jax.experimental import pallas as pl
from jax.experimental.pallas import tpu as pltpu
from jax.experimental.pallas import tpu_sc as plsc

VOCAB = 100000
MODEL_DIM = 128
BATCH = 4
SEQ_LEN = 8192

_NUM_WORKERS = 32
_CHUNK = SEQ_LEN // _NUM_WORKERS
_GATHER = 128
_G_PER_CHUNK = _CHUNK // _GATHER
_NBUF = 5
_N_CHUNKS_TOT = BATCH * _G_PER_CHUNK
_LANES = 16


def _frontend_body(x_hbm, emb_hbm, pos_hbm, out_hbm, idx_v, pos_v, acc_v,
                   sem_i, sem_st, *sems):
    sem_g = sems[0:_NBUF]
    sem_s = sems[_NBUF:2 * _NBUF]
    c = lax.axis_index("c")
    s = lax.axis_index("s")
    wid = s * 2 + c
    idx_cp = [
        pltpu.async_copy(
            x_hbm.at[b].at[pl.ds(wid * _CHUNK, _CHUNK)],
            idx_v.at[pl.ds(b * _CHUNK, _CHUNK)], sem_i)
        for b in range(BATCH)
    ]
    stage_cp = [pltpu.async_copy(
        pos_hbm.at[pl.ds(wid * _CHUNK, _CHUNK)], pos_v, sem_st)]

    def gather(ck, buf):
        return pltpu.async_copy(
            emb_hbm.at[idx_v.at[pl.ds(ck * _GATHER, _GATHER)]],
            acc_v.at[pl.ds(buf * _GATHER, _GATHER)], sem_g[buf])

    def add_pos(buf, j):
        @plsc.parallel_loop(0, _GATHER, unroll=4)
        def body(row):
            arow = buf * _GATHER + row
            prow = j * _GATHER + row
            for cp16 in range(MODEL_DIM // _LANES):
                sl = pl.ds(cp16 * _LANES, _LANES)
                acc_v[arow, sl] = acc_v[arow, sl] + pos_v[prow, sl]

    g_cp = [None] * _NBUF
    st_cp = [None] * _NBUF

    for cp in idx_cp:
        cp.wait()
    for ck in range(_NBUF):
        g_cp[ck] = gather(ck, ck)

    for r in range(_N_CHUNKS_TOT):
        rbuf = r % _NBUF
        g_cp[rbuf].wait()
        if stage_cp[0] is not None:
            stage_cp[0].wait()
            stage_cp[0] = None
        b, j = r // _G_PER_CHUNK, r % _G_PER_CHUNK
        add_pos(rbuf, j)
        st_cp[rbuf] = pltpu.async_copy(
            acc_v.at[pl.ds(rbuf * _GATHER, _GATHER)],
            out_hbm.at[b].at[pl.ds(wid * _CHUNK + j * _GATHER, _GATHER)],
            sem_s[rbuf])
        nxt = r + _NBUF
        if nxt < _N_CHUNKS_TOT:
            st_cp[rbuf].wait()
            g_cp[rbuf] = gather(nxt, rbuf)
    for cp in st_cp:
        if cp is not None:
            cp.wait()


@jax.jit
def kernel(x, embed_weight, pos_weight):
    mesh = plsc.VectorSubcoreMesh(core_axis_name="c", subcore_axis_name="s")
    return pl.kernel(
        _frontend_body,
        out_type=jax.ShapeDtypeStruct((BATCH, SEQ_LEN, MODEL_DIM), jnp.float32),
        mesh=mesh,
        scratch_types=[
            pltpu.VMEM((BATCH * _CHUNK,), jnp.int32),
            pltpu.VMEM((_CHUNK, MODEL_DIM), jnp.float32),
            pltpu.VMEM((_NBUF * _GATHER, MODEL_DIM), jnp.float32),
        ] + [pltpu.SemaphoreType.DMA] * (2 + 2 * _NBUF),
    )(x.astype(jnp.int32), embed_weight, pos_weight)


# speedup vs baseline: 1.0415x; 1.0415x over previous
import jax
import jax.numpy as jnp
from jax import lax
from jax.experimental import pallas as pl
from jax.experimental.pallas import tpu as pltpu
from jax.experimental.pallas import tpu_sc as plsc

VOCAB = 100000
MODEL_DIM = 128
BATCH = 4
SEQ_LEN = 8192

_NUM_WORKERS = 32
_CHUNK = SEQ_LEN // _NUM_WORKERS
_GATHER = 128
_G_PER_CHUNK = _CHUNK // _GATHER
_NBUF = 5
_N_CHUNKS_TOT = BATCH * _G_PER_CHUNK
_LANES = 16


def _frontend_body(x_hbm, emb_hbm, pos_hbm, out_hbm, idx_v, pos_v, acc_v,
                   sem_i, sem_st, *sems):
    sem_g = sems[0:_NBUF]
    sem_s = sems[_NBUF:2 * _NBUF]
    c = lax.axis_index("c")
    s = lax.axis_index("s")
    wid = s * 2 + c
    idx_cp = [
        pltpu.async_copy(
            x_hbm.at[b].at[pl.ds(wid * _CHUNK, _CHUNK)],
            idx_v.at[pl.ds(b * _CHUNK, _CHUNK)], sem_i)
        for b in range(BATCH)
    ]
    stage_cp = [pltpu.async_copy(
        pos_hbm.at[pl.ds(wid * _CHUNK, _CHUNK)], pos_v, sem_st)]

    def gather(ck, buf):
        return pltpu.async_copy(
            emb_hbm.at[idx_v.at[pl.ds(ck * _GATHER, _GATHER)]],
            acc_v.at[pl.ds(buf * _GATHER, _GATHER)], sem_g[buf])

    def add_pos(buf, j):
        def body(row, carry):
            arow = buf * _GATHER + row
            prow = j * _GATHER + row
            for cp16 in range(MODEL_DIM // _LANES):
                sl = pl.ds(cp16 * _LANES, _LANES)
                acc_v[arow, sl] = acc_v[arow, sl] + pos_v[prow, sl]
            return carry
        lax.fori_loop(0, _GATHER, body, 0)

    g_cp = [None] * _NBUF
    st_cp = [None] * _NBUF

    for cp in idx_cp:
        cp.wait()
    for ck in range(_NBUF):
        g_cp[ck] = gather(ck, ck)

    pending = []
    for r in range(_N_CHUNKS_TOT):
        rbuf = r % _NBUF
        for (nxt, nbuf) in pending:
            st_cp[nbuf].wait()
            g_cp[nbuf] = gather(nxt, nbuf)
        pending = []
        g_cp[rbuf].wait()
        if stage_cp[0] is not None:
            stage_cp[0].wait()
            stage_cp[0] = None
        b, j = r // _G_PER_CHUNK, r % _G_PER_CHUNK
        add_pos(rbuf, j)
        st_cp[rbuf] = pltpu.async_copy(
            acc_v.at[pl.ds(rbuf * _GATHER, _GATHER)],
            out_hbm.at[b].at[pl.ds(wid * _CHUNK + j * _GATHER, _GATHER)],
            sem_s[rbuf])
        nxt = r + _NBUF
        if nxt < _N_CHUNKS_TOT:
            pending.append((nxt, rbuf))
    for cp in st_cp:
        if cp is not None:
            cp.wait()


@jax.jit
def kernel(x, embed_weight, pos_weight):
    mesh = plsc.VectorSubcoreMesh(core_axis_name="c", subcore_axis_name="s")
    return pl.kernel(
        _frontend_body,
        out_type=jax.ShapeDtypeStruct((BATCH, SEQ_LEN, MODEL_DIM), jnp.float32),
        mesh=mesh,
        scratch_types=[
            pltpu.VMEM((BATCH * _CHUNK,), jnp.int32),
            pltpu.VMEM((_CHUNK, MODEL_DIM), jnp.float32),
            pltpu.VMEM((_NBUF * _GATHER, MODEL_DIM), jnp.float32),
        ] + [pltpu.SemaphoreType.DMA] * (2 + 2 * _NBUF),
    )(x.astype(jnp.int32), embed_weight, pos_weight)


# speedup vs baseline: 1.0589x; 1.0167x over previous
import jax
import jax.numpy as jnp
from jax import lax
from jax.experimental import pallas as pl
from jax.experimental.pallas import tpu as pltpu
from jax.experimental.pallas import tpu_sc as plsc

VOCAB = 100000
MODEL_DIM = 128
BATCH = 4
SEQ_LEN = 8192

_NUM_WORKERS = 32
_CHUNK = SEQ_LEN // _NUM_WORKERS
_GATHER = 128
_G_PER_CHUNK = _CHUNK // _GATHER
_NBUF = 5


_N_CHUNKS_TOT = BATCH * _G_PER_CHUNK


def _frontend_body(x_hbm, emb_hbm, pos_hbm, out_hbm, idx_v, pos_sh, acc_v,
                   sem_i, sem_st, *sems):
    sem_p = sems[0:_NBUF]
    sem_g = sems[_NBUF:2 * _NBUF]
    sem_s = sems[2 * _NBUF:3 * _NBUF]
    c = lax.axis_index("c")
    s = lax.axis_index("s")
    wid = s * 2 + c
    idx_cp = [
        pltpu.async_copy(
            x_hbm.at[b].at[pl.ds(wid * _CHUNK, _CHUNK)],
            idx_v.at[pl.ds(b * _CHUNK, _CHUNK)], sem_i)
        for b in range(BATCH)
    ]
    stage_cp = [pltpu.async_copy(
        pos_hbm.at[pl.ds(wid * _CHUNK, _CHUNK)], pos_sh.at[s], sem_st)]

    def pos_load(ck, buf):
        j = ck % _G_PER_CHUNK
        if ck < _NBUF:
            return pltpu.async_copy(
                pos_hbm.at[pl.ds(wid * _CHUNK + j * _GATHER, _GATHER)],
                acc_v.at[buf], sem_p[buf])
        if stage_cp[0] is not None:
            stage_cp[0].wait()
            stage_cp[0] = None
        return pltpu.async_copy(
            pos_sh.at[s].at[pl.ds(j * _GATHER, _GATHER)],
            acc_v.at[buf], sem_p[buf])

    pos_cp = [None] * _NBUF
    g_cp = [None] * _NBUF
    st_cp = [None] * _NBUF

    pending = []

    def retire(r):
        rbuf = r % _NBUF
        g_cp[rbuf].wait()
        b, j = r // _G_PER_CHUNK, r % _G_PER_CHUNK
        st_cp[rbuf] = pltpu.async_copy(
            acc_v.at[rbuf],
            out_hbm.at[b].at[pl.ds(wid * _CHUNK + j * _GATHER, _GATHER)],
            sem_s[rbuf])
        nxt = r + _NBUF
        if nxt < _N_CHUNKS_TOT:
            pending.append((nxt, rbuf))

    for ck in range(_NBUF):
        pos_cp[ck] = pos_load(ck, ck)
    for cp in idx_cp:
        cp.wait()
    for ck in range(_N_CHUNKS_TOT):
        buf = ck % _NBUF
        for (nxt, nbuf) in pending:
            st_cp[nbuf].wait()
            pos_cp[nbuf] = pos_load(nxt, nbuf)
        pending.clear()
        pos_cp[buf].wait()
        g_cp[buf] = pltpu.async_copy(
            emb_hbm.at[idx_v.at[pl.ds(ck * _GATHER, _GATHER)]],
            acc_v.at[buf], sem_g[buf], add=True)
        if ck - (_NBUF - 1) >= 0:
            retire(ck - (_NBUF - 1))
    for r in range(_N_CHUNKS_TOT - _NBUF + 1, _N_CHUNKS_TOT):
        retire(r)
    for cp in st_cp:
        if cp is not None:
            cp.wait()


@jax.jit
def kernel(x, embed_weight, pos_weight):
    mesh = plsc.VectorSubcoreMesh(core_axis_name="c", subcore_axis_name="s")
    return pl.kernel(
        _frontend_body,
        out_type=jax.ShapeDtypeStruct((BATCH, SEQ_LEN, MODEL_DIM), jnp.float32),
        mesh=mesh,
        scratch_types=[
            pltpu.VMEM((BATCH * _CHUNK,), jnp.int32),
            pltpu.VMEM_SHARED((16, _CHUNK, MODEL_DIM), jnp.float32),
            pltpu.VMEM((_NBUF, _GATHER, MODEL_DIM), jnp.float32),
        ] + [pltpu.SemaphoreType.DMA] * (2 + 3 * _NBUF),
    )(x.astype(jnp.int32), embed_weight, pos_weight)
